# baseline (speedup 1.0000x reference)
import functools

import jax
import jax.numpy as jnp
from jax import lax
from jax.experimental import pallas as pl
from jax.experimental.pallas import tpu as pltpu
from jax.experimental.pallas import tpu_sc as plsc

NUM_DATASETS = 1000
PROMPT_LEN = 20
HIDDEN = 2048
BATCH = 1024

_info = plsc.get_sparse_core_info()
NC, NS, L = _info.num_cores, _info.num_subcores, _info.num_lanes
NW = NC * NS

B_PER_W = BATCH // NW
HALVES = B_PER_W // L


@functools.partial(
    pl.kernel,
    out_type=jax.ShapeDtypeStruct((BATCH, PROMPT_LEN, HIDDEN), jnp.float32),
    mesh=plsc.VectorSubcoreMesh(core_axis_name="c", subcore_axis_name="s"),
    scratch_types=[
        pltpu.VMEM((B_PER_W,), jnp.int32),
        pltpu.VMEM((L,), jnp.int32),
        pltpu.VMEM((L, HIDDEN), jnp.float32),
        pltpu.SemaphoreType.DMA,
    ],
)
def _gather_kernel(ds_hbm, tab_hbm, out_hbm, ds_v, idx_v, buf, sem):
    wid = lax.axis_index("s") * NC + lax.axis_index("c")
    b0 = wid * B_PER_W
    pltpu.sync_copy(ds_hbm.at[pl.ds(b0, B_PER_W)], ds_v)

    for h in range(HALVES):
        ids = ds_v[pl.ds(h * L, L)] * PROMPT_LEN

        def jstep(j, carry, ids=ids, h=h):
            idx_v[...] = ids + j
            pltpu.async_copy(tab_hbm.at[idx_v], buf, sem).wait()
            pltpu.sync_copy(buf, out_hbm.at[pl.ds(b0 + h * L, L), j])
            return carry

        lax.fori_loop(0, PROMPT_LEN, jstep, 0)


def kernel(ds_ids, prompts):
    tab = prompts.reshape(NUM_DATASETS * PROMPT_LEN, HIDDEN)
    out = _gather_kernel(ds_ids.astype(jnp.int32), tab)
    return out


# speedup vs baseline: 1.0142x; 1.0142x over previous
import functools

import jax
import jax.numpy as jnp
from jax import lax
from jax.experimental import pallas as pl
from jax.experimental.pallas import tpu as pltpu
from jax.experimental.pallas import tpu_sc as plsc

NUM_DATASETS = 1000
PROMPT_LEN = 20
HIDDEN = 2048
BATCH = 1024

_info = plsc.get_sparse_core_info()
NC, NS, L = _info.num_cores, _info.num_subcores, _info.num_lanes
NW = NC * NS

B_PER_W = BATCH // NW
HALVES = B_PER_W // L
STEPS = HALVES * PROMPT_LEN


@functools.partial(
    pl.kernel,
    out_type=jax.ShapeDtypeStruct((BATCH, PROMPT_LEN, HIDDEN), jnp.float32),
    mesh=plsc.VectorSubcoreMesh(core_axis_name="c", subcore_axis_name="s"),
    scratch_types=[
        pltpu.VMEM((B_PER_W,), jnp.int32),
        pltpu.VMEM((L,), jnp.int32),
        pltpu.VMEM((L,), jnp.int32),
        pltpu.VMEM((L, HIDDEN), jnp.float32),
        pltpu.VMEM((L, HIDDEN), jnp.float32),
        pltpu.SemaphoreType.DMA,
        pltpu.SemaphoreType.DMA,
        pltpu.SemaphoreType.DMA,
        pltpu.SemaphoreType.DMA,
    ],
)
def _gather_kernel(ds_hbm, tab_hbm, out_hbm, ds_v, idx0, idx1, buf0, buf1,
                   g0, g1, w0, w1):
    wid = lax.axis_index("s") * NC + lax.axis_index("c")
    b0 = wid * B_PER_W
    pltpu.sync_copy(ds_hbm.at[pl.ds(b0, B_PER_W)], ds_v)

    idxs = (idx0, idx1)
    bufs = (buf0, buf1)
    gsems = (g0, g1)
    wsems = (w0, w1)

    rows0 = ds_v[pl.ds(0, L)] * PROMPT_LEN
    rows1 = ds_v[pl.ds(L, L)] * PROMPT_LEN

    def halfpos(t):
        in_hi = t >= PROMPT_LEN
        h = lax.select(in_hi, 1, 0)
        j = t - h * PROMPT_LEN
        return h, j

    def g_issue(t, slot):
        h, j = halfpos(t)
        base = rows0 + (rows1 - rows0) * h
        idxs[slot][...] = base + j
        pltpu.async_copy(tab_hbm.at[idxs[slot]], bufs[slot], gsems[slot])

    def w_issue(t, slot):
        h, j = halfpos(t)
        pltpu.async_copy(
            bufs[slot], out_hbm.at[pl.ds(b0 + h * L, L), j], wsems[slot])

    def g_wait(slot):
        pltpu.make_async_copy(
            tab_hbm.at[idxs[slot]], bufs[slot], gsems[slot]).wait()

    def w_wait(slot):
        pltpu.make_async_copy(
            bufs[slot], out_hbm.at[pl.ds(0, L), 0], wsems[slot]).wait()

    g_issue(0, 0)
    g_wait(0)
    w_issue(0, 0)
    g_issue(1, 1)

    def step(t, carry):
        s = lax.rem(t, 2)

        @pl.when(s == 0)
        def _():
            g_wait(0)
            w_issue(t, 0)
            w_wait(1)
            g_issue(t + 1, 1)

        @pl.when(s == 1)
        def _():
            g_wait(1)
            w_issue(t, 1)
            w_wait(0)
            g_issue(t + 1, 0)

        return carry

    lax.fori_loop(1, STEPS - 1, step, 0)

    g_wait(1)
    w_issue(STEPS - 1, 1)
    w_wait(0)
    w_wait(1)


def kernel(ds_ids, prompts):
    tab = prompts.reshape(NUM_DATASETS * PROMPT_LEN, HIDDEN)
    out = _gather_kernel(ds_ids.astype(jnp.int32), tab)
    return out


# speedup vs baseline: 3.0106x; 2.9685x over previous
import functools

import jax
import jax.numpy as jnp
from jax import lax
from jax.experimental import pallas as pl
from jax.experimental.pallas import tpu as pltpu
from jax.experimental.pallas import tpu_sc as plsc

NUM_DATASETS = 1000
PROMPT_LEN = 20
HIDDEN = 2048
BATCH = 1024

_info = plsc.get_sparse_core_info()
NC, NS, L = _info.num_cores, _info.num_subcores, _info.num_lanes
NW = NC * NS

B_PER_W = BATCH // NW


@functools.partial(
    pl.kernel,
    out_type=jax.ShapeDtypeStruct((BATCH, PROMPT_LEN, HIDDEN), jnp.float32),
    mesh=plsc.VectorSubcoreMesh(core_axis_name="c", subcore_axis_name="s"),
    scratch_types=[
        pltpu.SMEM((B_PER_W,), jnp.int32),
        pltpu.VMEM((B_PER_W,), jnp.int32),
        pltpu.VMEM_SHARED((NS, B_PER_W), jnp.int32),
        pltpu.VMEM((PROMPT_LEN, HIDDEN), jnp.float32),
        pltpu.VMEM((PROMPT_LEN, HIDDEN), jnp.float32),
        pltpu.SemaphoreType.DMA,
        pltpu.SemaphoreType.DMA,
        pltpu.SemaphoreType.DMA,
        pltpu.SemaphoreType.DMA,
    ],
)
def _gather_kernel(ds_hbm, tab_hbm, out_hbm, ids_s, ids_v, ids_sh,
                   buf0, buf1, g0, g1, w0, w1):
    wid = lax.axis_index("s") * NC + lax.axis_index("c")
    b0 = wid * B_PER_W
    sid = lax.axis_index("s")
    pltpu.sync_copy(ds_hbm.at[pl.ds(b0, B_PER_W)], ids_v)
    pltpu.sync_copy(ids_v, ids_sh.at[sid])
    pltpu.sync_copy(ids_sh.at[sid], ids_s)

    bufs = (buf0, buf1)
    gsems = (g0, g1)
    wsems = (w0, w1)

    def g_issue(t, slot):
        pltpu.async_copy(tab_hbm.at[ids_s[t]], bufs[slot], gsems[slot])

    def w_issue(t, slot):
        pltpu.async_copy(bufs[slot], out_hbm.at[b0 + t], wsems[slot])

    def g_wait(slot):
        pltpu.make_async_copy(tab_hbm.at[0], bufs[slot], gsems[slot]).wait()

    def w_wait(slot):
        pltpu.make_async_copy(bufs[slot], out_hbm.at[0], wsems[slot]).wait()

    g_issue(0, 0)
    g_wait(0)
    w_issue(0, 0)
    g_issue(1, 1)

    def step(t, carry):
        s = lax.rem(t, 2)

        @pl.when(s == 0)
        def _():
            g_wait(0)
            w_issue(t, 0)
            w_wait(1)
            g_issue(t + 1, 1)

        @pl.when(s == 1)
        def _():
            g_wait(1)
            w_issue(t, 1)
            w_wait(0)
            g_issue(t + 1, 0)

        return carry

    lax.fori_loop(1, B_PER_W - 1, step, 0)

    g_wait(1)
    w_issue(B_PER_W - 1, 1)
    w_wait(0)
    w_wait(1)


def kernel(ds_ids, prompts):
    return _gather_kernel(ds_ids.astype(jnp.int32), prompts)


# speedup vs baseline: 3.0139x; 1.0011x over previous
import functools

import jax
import jax.numpy as jnp
from jax import lax
from jax.experimental import pallas as pl
from jax.experimental.pallas import tpu as pltpu
from jax.experimental.pallas import tpu_sc as plsc

NUM_DATASETS = 1000
PROMPT_LEN = 20
HIDDEN = 2048
BATCH = 1024

_info = plsc.get_sparse_core_info()
NC, NS, L = _info.num_cores, _info.num_subcores, _info.num_lanes
NW = NC * NS

B_PER_W = BATCH // NW


@functools.partial(
    pl.kernel,
    out_type=jax.ShapeDtypeStruct((BATCH, PROMPT_LEN, HIDDEN), jnp.float32),
    mesh=plsc.VectorSubcoreMesh(core_axis_name="c", subcore_axis_name="s"),
    compiler_params=pltpu.CompilerParams(use_tc_tiling_on_sc=True),
    scratch_types=[
        pltpu.SMEM((B_PER_W,), jnp.int32),
        pltpu.VMEM((B_PER_W,), jnp.int32),
        pltpu.VMEM_SHARED((NS, B_PER_W), jnp.int32),
        pltpu.VMEM((PROMPT_LEN, HIDDEN), jnp.float32),
        pltpu.VMEM((PROMPT_LEN, HIDDEN), jnp.float32),
        pltpu.SemaphoreType.DMA,
        pltpu.SemaphoreType.DMA,
        pltpu.SemaphoreType.DMA,
        pltpu.SemaphoreType.DMA,
    ],
)
def _gather_kernel(ds_hbm, tab_hbm, out_hbm, ids_s, ids_v, ids_sh,
                   buf0, buf1, g0, g1, w0, w1):
    wid = lax.axis_index("s") * NC + lax.axis_index("c")
    b0 = wid * B_PER_W
    sid = lax.axis_index("s")
    pltpu.sync_copy(ds_hbm.at[pl.ds(b0, B_PER_W)], ids_v)
    pltpu.sync_copy(ids_v, ids_sh.at[sid])
    pltpu.sync_copy(ids_sh.at[sid], ids_s)

    bufs = (buf0, buf1)
    gsems = (g0, g1)
    wsems = (w0, w1)

    def g_issue(t, slot):
        pltpu.async_copy(tab_hbm.at[ids_s[t]], bufs[slot], gsems[slot])

    def w_issue(t, slot):
        pltpu.async_copy(bufs[slot], out_hbm.at[b0 + t], wsems[slot])

    def g_wait(slot):
        pltpu.make_async_copy(tab_hbm.at[0], bufs[slot], gsems[slot]).wait()

    def w_wait(slot):
        pltpu.make_async_copy(bufs[slot], out_hbm.at[0], wsems[slot]).wait()

    g_issue(0, 0)
    g_wait(0)
    w_issue(0, 0)
    g_issue(1, 1)

    def step(t, carry):
        s = lax.rem(t, 2)

        @pl.when(s == 0)
        def _():
            g_wait(0)
            w_issue(t, 0)
            w_wait(1)
            g_issue(t + 1, 1)

        @pl.when(s == 1)
        def _():
            g_wait(1)
            w_issue(t, 1)
            w_wait(0)
            g_issue(t + 1, 0)

        return carry

    lax.fori_loop(1, B_PER_W - 1, step, 0)

    g_wait(1)
    w_issue(B_PER_W - 1, 1)
    w_wait(0)
    w_wait(1)


def kernel(ds_ids, prompts):
    return _gather_kernel(ds_ids.astype(jnp.int32), prompts)


# speedup vs baseline: 8.2738x; 2.7452x over previous
import functools

import jax
import jax.numpy as jnp
from jax import lax
from jax.experimental import pallas as pl
from jax.experimental.pallas import tpu as pltpu
from jax.experimental.pallas import tpu_sc as plsc

NUM_DATASETS = 1000
PROMPT_LEN = 20
HIDDEN = 2048
BATCH = 1024

_info = plsc.get_sparse_core_info()
NC, NS, L = _info.num_cores, _info.num_subcores, _info.num_lanes
NW = NC * NS

B_PER_W = BATCH // NW


@functools.partial(
    pl.kernel,
    out_type=jax.ShapeDtypeStruct((PROMPT_LEN, BATCH, HIDDEN), jnp.float32),
    mesh=plsc.VectorSubcoreMesh(core_axis_name="c", subcore_axis_name="s"),
    compiler_params=pltpu.CompilerParams(use_tc_tiling_on_sc=True),
    scratch_types=[
        pltpu.SMEM((B_PER_W,), jnp.int32),
        pltpu.VMEM((B_PER_W,), jnp.int32),
        pltpu.VMEM_SHARED((NS, B_PER_W), jnp.int32),
        pltpu.VMEM((PROMPT_LEN, HIDDEN), jnp.float32),
        pltpu.VMEM((PROMPT_LEN, HIDDEN), jnp.float32),
        pltpu.SemaphoreType.DMA,
        pltpu.SemaphoreType.DMA,
        pltpu.SemaphoreType.DMA,
        pltpu.SemaphoreType.DMA,
    ],
)
def _gather_kernel(ds_hbm, tab_hbm, out_hbm, ids_s, ids_v, ids_sh,
                   buf0, buf1, g0, g1, w0, w1):
    wid = lax.axis_index("s") * NC + lax.axis_index("c")
    b0 = wid * B_PER_W
    sid = lax.axis_index("s")
    pltpu.sync_copy(ds_hbm.at[pl.ds(b0, B_PER_W)], ids_v)
    pltpu.sync_copy(ids_v, ids_sh.at[sid])
    pltpu.sync_copy(ids_sh.at[sid], ids_s)

    bufs = (buf0, buf1)
    gsems = (g0, g1)
    wsems = (w0, w1)

    def g_issue(t, slot):
        pltpu.async_copy(tab_hbm.at[pl.ds(0, PROMPT_LEN), ids_s[t]],
                         bufs[slot], gsems[slot])

    def w_issue(t, slot):
        pltpu.async_copy(bufs[slot], out_hbm.at[pl.ds(0, PROMPT_LEN), b0 + t],
                         wsems[slot])

    def g_wait(slot):
        pltpu.make_async_copy(tab_hbm.at[pl.ds(0, PROMPT_LEN), 0],
                              bufs[slot], gsems[slot]).wait()

    def w_wait(slot):
        pltpu.make_async_copy(bufs[slot], out_hbm.at[pl.ds(0, PROMPT_LEN), 0],
                              wsems[slot]).wait()

    g_issue(0, 0)
    g_wait(0)
    w_issue(0, 0)
    g_issue(1, 1)

    def step(t, carry):
        s = lax.rem(t, 2)

        @pl.when(s == 0)
        def _():
            g_wait(0)
            w_issue(t, 0)
            w_wait(1)
            g_issue(t + 1, 1)

        @pl.when(s == 1)
        def _():
            g_wait(1)
            w_issue(t, 1)
            w_wait(0)
            g_issue(t + 1, 0)

        return carry

    lax.fori_loop(1, B_PER_W - 1, step, 0)

    g_wait(1)
    w_issue(B_PER_W - 1, 1)
    w_wait(0)
    w_wait(1)


def kernel(ds_ids, prompts):
    tab_t = prompts.transpose(1, 0, 2)
    out_t = _gather_kernel(ds_ids.astype(jnp.int32), tab_t)
    return out_t.transpose(1, 0, 2)


# speedup vs baseline: 8.3742x; 1.0121x over previous
import functools

import jax
import jax.numpy as jnp
from jax import lax
from jax.experimental import pallas as pl
from jax.experimental.pallas import tpu as pltpu
from jax.experimental.pallas import tpu_sc as plsc

NUM_DATASETS = 1000
PROMPT_LEN = 20
HIDDEN = 2048
BATCH = 1024

_info = plsc.get_sparse_core_info()
NC, NS, L = _info.num_cores, _info.num_subcores, _info.num_lanes
NW = NC * NS

B_PER_W = BATCH // NW


@functools.partial(
    pl.kernel,
    out_type=jax.ShapeDtypeStruct((PROMPT_LEN, BATCH, HIDDEN), jnp.float32),
    mesh=plsc.VectorSubcoreMesh(core_axis_name="c", subcore_axis_name="s"),
    compiler_params=pltpu.CompilerParams(use_tc_tiling_on_sc=True),
    scratch_types=[
        pltpu.SMEM((B_PER_W,), jnp.int32),
        pltpu.VMEM((B_PER_W,), jnp.int32),
        pltpu.VMEM_SHARED((NS, B_PER_W), jnp.int32),
        pltpu.VMEM((PROMPT_LEN, HIDDEN), jnp.float32),
        pltpu.VMEM((PROMPT_LEN, HIDDEN), jnp.float32),
        pltpu.SemaphoreType.DMA,
        pltpu.SemaphoreType.DMA,
        pltpu.SemaphoreType.DMA,
        pltpu.SemaphoreType.DMA,
    ],
)
def _gather_kernel(ds_hbm, tab_hbm, out_hbm, ids_s, ids_v, ids_sh,
                   buf0, buf1, g0, g1, w0, w1):
    wid = lax.axis_index("s") * NC + lax.axis_index("c")
    b0 = wid * B_PER_W
    sid = lax.axis_index("s")
    pltpu.sync_copy(ds_hbm.at[pl.ds(b0, B_PER_W)], ids_v)
    pltpu.sync_copy(ids_v, ids_sh.at[sid])
    pltpu.sync_copy(ids_sh.at[sid], ids_s)

    bufs = (buf0, buf1)
    gsems = (g0, g1)
    wsems = (w0, w1)

    H2 = HIDDEN // 2

    def g_issue(t, slot):
        for m in range(2):
            pltpu.async_copy(
                tab_hbm.at[pl.ds(0, PROMPT_LEN), ids_s[t], pl.ds(m * H2, H2)],
                bufs[slot].at[:, pl.ds(m * H2, H2)], gsems[slot])

    def w_issue(t, slot):
        for m in range(2):
            pltpu.async_copy(
                bufs[slot].at[:, pl.ds(m * H2, H2)],
                out_hbm.at[pl.ds(0, PROMPT_LEN), b0 + t, pl.ds(m * H2, H2)],
                wsems[slot])

    def g_wait(slot):
        pltpu.make_async_copy(tab_hbm.at[pl.ds(0, PROMPT_LEN), 0],
                              bufs[slot], gsems[slot]).wait()

    def w_wait(slot):
        pltpu.make_async_copy(bufs[slot], out_hbm.at[pl.ds(0, PROMPT_LEN), 0],
                              wsems[slot]).wait()

    g_issue(0, 0)
    g_wait(0)
    w_issue(0, 0)
    g_issue(1, 1)

    def step(t, carry):
        s = lax.rem(t, 2)

        @pl.when(s == 0)
        def _():
            g_wait(0)
            w_issue(t, 0)
            w_wait(1)
            g_issue(t + 1, 1)

        @pl.when(s == 1)
        def _():
            g_wait(1)
            w_issue(t, 1)
            w_wait(0)
            g_issue(t + 1, 0)

        return carry

    lax.fori_loop(1, B_PER_W - 1, step, 0)

    g_wait(1)
    w_issue(B_PER_W - 1, 1)
    w_wait(0)
    w_wait(1)


def kernel(ds_ids, prompts):
    tab_t = prompts.transpose(1, 0, 2)
    out_t = _gather_kernel(ds_ids.astype(jnp.int32), tab_t)
    return out_t.transpose(1, 0, 2)
